# baseline (speedup 1.0000x reference)
import functools

import jax
import jax.numpy as jnp
from jax import lax
from jax.experimental import pallas as pl
from jax.experimental.pallas import tpu as pltpu


def _lstm_kernel(u_ref, h0_ref, c0_ref, wih_ref, whh_ref, b_ref,
                 states_ref, h_sc, c_sc):
    tb = pl.program_id(0)

    @pl.when(tb == 0)
    def _():
        h_sc[...] = h0_ref[...]
        c_sc[...] = c0_ref[...]

    nx = h_sc.shape[0]
    n_groups = u_ref.shape[1] // 8

    def group_step(g, carry):
        hT, cT = carry
        base = pl.multiple_of(g * 8, 8)
        hs = []
        for j in range(8):
            u_t = u_ref[:, base + j, :]
            gxT = lax.dot_general(
                wih_ref[...], u_t, (((1,), (1,)), ((), ())),
                preferred_element_type=jnp.float32) + b_ref[...]
            gatesT = gxT + lax.dot_general(
                whh_ref[...], hT, (((1,), (0,)), ((), ())),
                preferred_element_type=jnp.float32)
            t = jnp.tanh(gatesT)
            si = t[0 * nx:1 * nx] * 0.5 + 0.5
            sf = t[1 * nx:2 * nx] * 0.5 + 0.5
            tg = t[2 * nx:3 * nx]
            so = t[3 * nx:4 * nx] * 0.5 + 0.5
            cT = sf * cT + si * tg
            hT = so * jnp.tanh(cT)
            hs.append(hT)
        states_ref[2 * g] = jnp.concatenate(hs[:4], axis=0)
        states_ref[2 * g + 1] = jnp.concatenate(hs[4:], axis=0)
        return (hT, cT)

    h_fin, c_fin = lax.fori_loop(0, n_groups, group_step,
                                 (h_sc[...], c_sc[...]), unroll=2)
    h_sc[...] = h_fin
    c_sc[...] = c_fin


@functools.partial(jax.jit, static_argnames=("t_blk",))
def _lstm_forward(u, h0, c0, w_ih, w_hh, b_ih, b_hh, w_out, b_out, *,
                  t_blk=128):
    B, T, nu = u.shape
    nx = w_hh.shape[1]
    n_chunks = T // t_blk

    s = jnp.concatenate([jnp.full((2 * nx,), 0.5), jnp.ones((nx,)),
                         jnp.full((nx,), 0.5)]).astype(jnp.float32)
    wih_s = w_ih.astype(jnp.float32) * s[:, None]
    whh_s = w_hh.astype(jnp.float32) * s[:, None]
    bias_bc = jnp.broadcast_to(
        ((b_ih + b_hh).astype(jnp.float32) * s)[:, None], (4 * nx, B))
    h0_t = h0[0].T.astype(jnp.float32)
    c0_t = c0[0].T.astype(jnp.float32)

    states_packed = pl.pallas_call(
        _lstm_kernel,
        out_shape=jax.ShapeDtypeStruct((T // 4, 4 * nx, B), jnp.float32),
        grid=(n_chunks,),
        in_specs=[
            pl.BlockSpec((B, t_blk, nu), lambda tb: (0, tb, 0)),
            pl.BlockSpec((nx, B), lambda tb: (0, 0)),
            pl.BlockSpec((nx, B), lambda tb: (0, 0)),
            pl.BlockSpec((4 * nx, nu), lambda tb: (0, 0)),
            pl.BlockSpec((4 * nx, nx), lambda tb: (0, 0)),
            pl.BlockSpec((4 * nx, B), lambda tb: (0, 0)),
        ],
        out_specs=pl.BlockSpec((t_blk // 4, 4 * nx, B),
                               lambda tb: (tb, 0, 0)),
        scratch_shapes=[
            pltpu.VMEM((nx, B), jnp.float32),
            pltpu.VMEM((nx, B), jnp.float32),
        ],
        compiler_params=pltpu.CompilerParams(
            dimension_semantics=("arbitrary",),
        ),
    )(u, h0_t, c0_t, wih_s, whh_s, bias_bc)

    states = (states_packed.reshape(T // 4, 4, nx, B)
              .transpose(3, 0, 1, 2)
              .reshape(B, T, nx))

    y = jnp.einsum("btx,yx->bty", states, w_out) + b_out
    return y, states


def kernel(u, h0, c0, w_ih, w_hh, b_ih, b_hh, w_out, b_out):
    return _lstm_forward(u, h0, c0, w_ih, w_hh, b_ih, b_hh, w_out, b_out,
                         t_blk=64)


# speedup vs baseline: 1.0700x; 1.0700x over previous
import functools

import jax
import jax.numpy as jnp
from jax import lax
from jax.experimental import pallas as pl
from jax.experimental.pallas import tpu as pltpu


def _lstm_kernel(u_ref, h0_ref, c0_ref, wih_ref, whh_ref, b_ref,
                 states_ref, h_sc, c_sc):
    tb = pl.program_id(0)

    @pl.when(tb == 0)
    def _():
        h_sc[...] = h0_ref[...]
        c_sc[...] = c0_ref[...]

    nx = h_sc.shape[0]
    n_groups = u_ref.shape[1] // 8

    def group_step(g, carry):
        hT, cT = carry
        base = pl.multiple_of(g * 8, 8)
        hs = []
        for j in range(8):
            u_t = u_ref[:, base + j, :]
            gxT = lax.dot_general(
                wih_ref[...], u_t, (((1,), (1,)), ((), ())),
                preferred_element_type=jnp.float32) + b_ref[...]
            gatesT = gxT + lax.dot_general(
                whh_ref[...], hT, (((1,), (0,)), ((), ())),
                preferred_element_type=jnp.float32)
            t = jnp.tanh(gatesT)
            si = t[0 * nx:1 * nx] * 0.5 + 0.5
            sf = t[1 * nx:2 * nx] * 0.5 + 0.5
            tg = t[2 * nx:3 * nx]
            so = t[3 * nx:4 * nx] * 0.5 + 0.5
            cT = sf * cT + si * tg
            hT = so * jnp.tanh(cT)
            hs.append(hT)
        states_ref[2 * g] = jnp.concatenate(hs[:4], axis=0).astype(jnp.bfloat16)
        states_ref[2 * g + 1] = jnp.concatenate(hs[4:], axis=0).astype(jnp.bfloat16)
        return (hT, cT)

    h_fin, c_fin = lax.fori_loop(0, n_groups, group_step,
                                 (h_sc[...], c_sc[...]), unroll=2)
    h_sc[...] = h_fin
    c_sc[...] = c_fin


@functools.partial(jax.jit, static_argnames=("t_blk",))
def _lstm_forward(u, h0, c0, w_ih, w_hh, b_ih, b_hh, w_out, b_out, *,
                  t_blk=128):
    B, T, nu = u.shape
    nx = w_hh.shape[1]
    n_chunks = T // t_blk

    s = jnp.concatenate([jnp.full((2 * nx,), 0.5), jnp.ones((nx,)),
                         jnp.full((nx,), 0.5)]).astype(jnp.float32)
    wih_s = w_ih.astype(jnp.float32) * s[:, None]
    whh_s = w_hh.astype(jnp.float32) * s[:, None]
    bias_bc = jnp.broadcast_to(
        ((b_ih + b_hh).astype(jnp.float32) * s)[:, None], (4 * nx, B))
    h0_t = h0[0].T.astype(jnp.float32)
    c0_t = c0[0].T.astype(jnp.float32)

    states_packed = pl.pallas_call(
        _lstm_kernel,
        out_shape=jax.ShapeDtypeStruct((T // 4, 4 * nx, B), jnp.bfloat16),
        grid=(n_chunks,),
        in_specs=[
            pl.BlockSpec((B, t_blk, nu), lambda tb: (0, tb, 0)),
            pl.BlockSpec((nx, B), lambda tb: (0, 0)),
            pl.BlockSpec((nx, B), lambda tb: (0, 0)),
            pl.BlockSpec((4 * nx, nu), lambda tb: (0, 0)),
            pl.BlockSpec((4 * nx, nx), lambda tb: (0, 0)),
            pl.BlockSpec((4 * nx, B), lambda tb: (0, 0)),
        ],
        out_specs=pl.BlockSpec((t_blk // 4, 4 * nx, B),
                               lambda tb: (tb, 0, 0)),
        scratch_shapes=[
            pltpu.VMEM((nx, B), jnp.float32),
            pltpu.VMEM((nx, B), jnp.float32),
        ],
        compiler_params=pltpu.CompilerParams(
            dimension_semantics=("arbitrary",),
        ),
    )(u, h0_t, c0_t, wih_s, whh_s, bias_bc)

    states_bf = (states_packed.reshape(T // 4, 4, nx, B)
                 .transpose(3, 0, 1, 2)
                 .reshape(B, T, nx))
    states = states_bf.astype(jnp.float32)

    y = jnp.einsum("btx,yx->bty", states_bf, w_out.astype(jnp.bfloat16),
                   preferred_element_type=jnp.float32) + b_out
    return y, states


def kernel(u, h0, c0, w_ih, w_hh, b_ih, b_hh, w_out, b_out):
    return _lstm_forward(u, h0, c0, w_ih, w_hh, b_ih, b_hh, w_out, b_out,
                         t_blk=64)
